# baseline (speedup 1.0000x reference)
import functools

import jax
import jax.numpy as jnp
from jax import lax
from jax.experimental import pallas as pl
from jax.experimental.pallas import tpu as pltpu
from jax.experimental.pallas import tpu_sc as plsc

EMB = 64
BATCH = 16384
NC = 2
NS = 16
NW = NC * NS
PER_W = BATCH // NW
CHUNK = 128
NCHUNK = PER_W // CHUNK
GROUPS = CHUNK // 16


def _body(s_hbm, o_hbm, r_hbm, t_hbm, e_hbm, re_hbm, te_hbm, out_hbm,
          s_idx, o_idx, r_idx, t_idx, s_rows, r_rows, t_rows, o_rows,
          res, sem):
    wid = lax.axis_index("s") * NC + lax.axis_index("c")

    pltpu.sync_copy(s_hbm.at[wid], s_idx)
    pltpu.sync_copy(o_hbm.at[wid], o_idx)
    pltpu.sync_copy(r_hbm.at[wid], r_idx)
    pltpu.sync_copy(t_hbm.at[wid], t_idx)

    iota = lax.iota(jnp.int32, 16)

    for ch in range(NCHUNK):
        cs = pltpu.async_copy(e_hbm.at[s_idx.at[ch]], s_rows, sem)
        cr = pltpu.async_copy(re_hbm.at[r_idx.at[ch]], r_rows, sem)
        ct = pltpu.async_copy(te_hbm.at[t_idx.at[ch]], t_rows, sem)
        co = pltpu.async_copy(e_hbm.at[o_idx.at[ch]], o_rows, sem)
        cs.wait()
        cr.wait()
        ct.wait()
        co.wait()

        for g in range(GROUPS):
            rid = iota + (g * 16)

            def col_body(j, carry):
                acc, cj = carry
                va = plsc.load_gather(s_rows, [rid, cj])
                vb = plsc.load_gather(r_rows, [rid, cj])
                vc = plsc.load_gather(t_rows, [rid, cj])
                vd = plsc.load_gather(o_rows, [rid, cj])
                return acc + jnp.abs(va + vb + vc - vd), cj + 1

            acc, _ = lax.fori_loop(
                0, EMB, col_body,
                (jnp.zeros((16,), jnp.float32), jnp.zeros((16,), jnp.int32)))
            res[pl.ds(ch * CHUNK + g * 16, 16)] = acc

    pltpu.sync_copy(res, out_hbm.at[wid])


@jax.jit
def kernel(s, o, r, t, e_embed, r_embed, t_embed):
    s4 = s.astype(jnp.int32).reshape(NW, NCHUNK, CHUNK)
    o4 = o.astype(jnp.int32).reshape(NW, NCHUNK, CHUNK)
    r4 = r.astype(jnp.int32).reshape(NW, NCHUNK, CHUNK)
    t4 = t.astype(jnp.int32).reshape(NW, NCHUNK, CHUNK)

    mesh = plsc.VectorSubcoreMesh(core_axis_name="c", subcore_axis_name="s")
    run = pl.kernel(
        _body,
        out_type=jax.ShapeDtypeStruct((NW, PER_W), jnp.float32),
        mesh=mesh,
        compiler_params=pltpu.CompilerParams(needs_layout_passes=False, use_tc_tiling_on_sc=False),
        scratch_types=[
            pltpu.VMEM((NCHUNK, CHUNK), jnp.int32),
            pltpu.VMEM((NCHUNK, CHUNK), jnp.int32),
            pltpu.VMEM((NCHUNK, CHUNK), jnp.int32),
            pltpu.VMEM((NCHUNK, CHUNK), jnp.int32),
            pltpu.VMEM((CHUNK, EMB), jnp.float32),
            pltpu.VMEM((CHUNK, EMB), jnp.float32),
            pltpu.VMEM((CHUNK, EMB), jnp.float32),
            pltpu.VMEM((CHUNK, EMB), jnp.float32),
            pltpu.VMEM((PER_W,), jnp.float32),
            pltpu.SemaphoreType.DMA,
        ],
    )
    out = run(s4, o4, r4, t4, e_embed, r_embed, t_embed)
    return out.reshape(BATCH)


# speedup vs baseline: 1.1009x; 1.1009x over previous
import jax
import jax.numpy as jnp
from jax import lax
from jax.experimental import pallas as pl
from jax.experimental.pallas import tpu as pltpu
from jax.experimental.pallas import tpu_sc as plsc

EMB = 64
BATCH = 16384
NC = 2
NS = 16
NW = NC * NS
PER_W = BATCH // NW
CHUNK = 128
NCHUNK = PER_W // CHUNK
GROUPS = CHUNK // 16


def _body(s_hbm, o_hbm, r_hbm, t_hbm, e_hbm, re_hbm, te_hbm, out_hbm,
          s_idx, o_idx, r_idx, t_idx, acc0, acc1, ob0, ob1,
          res, sem_a0, sem_a1, sem_o0, sem_o1):
    wid = lax.axis_index("s") * NC + lax.axis_index("c")

    pltpu.sync_copy(s_hbm.at[wid], s_idx)
    pltpu.sync_copy(o_hbm.at[wid], o_idx)
    pltpu.sync_copy(r_hbm.at[wid], r_idx)
    pltpu.sync_copy(t_hbm.at[wid], t_idx)

    accs = (acc0, acc1)
    obs = (ob0, ob1)
    sems_a = (sem_a0, sem_a1)
    sems_o = (sem_o0, sem_o1)

    iota = lax.iota(jnp.int32, 16)

    def fire_ro(ch, b):
        cr = pltpu.async_copy(re_hbm.at[r_idx.at[ch]], accs[b], sems_a[b])
        co = pltpu.async_copy(e_hbm.at[o_idx.at[ch]], obs[b], sems_o[b])
        return cr, co

    def fire_t(ch, b):
        return pltpu.async_copy(te_hbm.at[t_idx.at[ch]], accs[b], sems_a[b],
                                add=True)

    def fire_s(ch, b):
        return pltpu.async_copy(e_hbm.at[s_idx.at[ch]], accs[b], sems_a[b],
                                add=True)

    def group(ch, b, g):
        rid = iota + (g * 16)

        def col_body(j, carry):
            acc, col = carry
            va = plsc.load_gather(accs[b], [rid, col])
            vo = plsc.load_gather(obs[b], [rid, col])
            return acc + jnp.abs(va - vo), (col + 1) & 63

        (acc, _) = plsc.parallel_loop(
            0, EMB, carry=(jnp.zeros((16,), jnp.float32), iota),
            unroll=8)(col_body)
        res[pl.ds(ch * CHUNK + g * 16, 16)] = acc

    cr, co = fire_ro(0, 0)
    cr.wait()
    fire_t(0, 0).wait()
    fire_s(0, 0).wait()
    co.wait()

    b = 0
    for ch in range(NCHUNK):
        nxt = ch + 1
        nb = 1 - b
        if nxt < NCHUNK:
            crn, con = fire_ro(nxt, nb)
        group(ch, b, 0)
        group(ch, b, 1)
        group(ch, b, 2)
        if nxt < NCHUNK:
            crn.wait()
            ctn = fire_t(nxt, nb)
        group(ch, b, 3)
        group(ch, b, 4)
        group(ch, b, 5)
        if nxt < NCHUNK:
            ctn.wait()
            csn = fire_s(nxt, nb)
        group(ch, b, 6)
        group(ch, b, 7)
        if nxt < NCHUNK:
            csn.wait()
            con.wait()
        b = nb

    pltpu.sync_copy(res, out_hbm.at[wid])


@jax.jit
def kernel(s, o, r, t, e_embed, r_embed, t_embed):
    s4 = s.astype(jnp.int32).reshape(NW, NCHUNK, CHUNK)
    o4 = o.astype(jnp.int32).reshape(NW, NCHUNK, CHUNK)
    r4 = r.astype(jnp.int32).reshape(NW, NCHUNK, CHUNK)
    t4 = t.astype(jnp.int32).reshape(NW, NCHUNK, CHUNK)

    mesh = plsc.VectorSubcoreMesh(core_axis_name="c", subcore_axis_name="s")
    run = pl.kernel(
        _body,
        out_type=jax.ShapeDtypeStruct((NW, PER_W), jnp.float32),
        mesh=mesh,
        compiler_params=pltpu.CompilerParams(
            needs_layout_passes=False, use_tc_tiling_on_sc=False),
        scratch_types=[
            pltpu.VMEM((NCHUNK, CHUNK), jnp.int32),
            pltpu.VMEM((NCHUNK, CHUNK), jnp.int32),
            pltpu.VMEM((NCHUNK, CHUNK), jnp.int32),
            pltpu.VMEM((NCHUNK, CHUNK), jnp.int32),
            pltpu.VMEM((CHUNK, EMB), jnp.float32),
            pltpu.VMEM((CHUNK, EMB), jnp.float32),
            pltpu.VMEM((CHUNK, EMB), jnp.float32),
            pltpu.VMEM((CHUNK, EMB), jnp.float32),
            pltpu.VMEM((PER_W,), jnp.float32),
            pltpu.SemaphoreType.DMA,
            pltpu.SemaphoreType.DMA,
            pltpu.SemaphoreType.DMA,
            pltpu.SemaphoreType.DMA,
        ],
    )
    out = run(s4, o4, r4, t4, e_embed, r_embed, t_embed)
    return out.reshape(BATCH)
